# baseline (speedup 1.0000x reference)
import functools
import jax
import jax.numpy as jnp
from jax import lax
from jax.experimental import pallas as pl
from jax.experimental.pallas import tpu as pltpu
from jax.experimental.pallas import tpu_sc as plsc

N_NODES = 10000
D = 128
E_EDGES = 320000
ROW_BLK = 1000

NCB = 8
NQ = 4
EPQ = E_EDGES // NQ
CE = 640
NCHUNK = EPQ // CE
NH = 5008
NEG_INF = float("-inf")


def _segmax_body(xcb_hbm, src_hbm, dst_hbm, out_hbm,
                 acc, srcb, dstb, idxb, offb, rows, sem):
    cid = lax.axis_index("c")
    sid = lax.axis_index("s")
    wid = sid * 2 + cid
    cb = wid % NCB
    q = wid // NCB

    for p in range(2):
        @pl.loop(0, NH + 1)
        def _init(i):
            acc[i, :] = jnp.full((16,), NEG_INF, jnp.float32)

        @pl.loop(0, NCHUNK)
        def _chunk(c):
            base_e = q * EPQ + c * CE
            pltpu.sync_copy(src_hbm.at[pl.ds(base_e, CE)], srcb)
            pltpu.sync_copy(dst_hbm.at[pl.ds(base_e, CE)], dstb)

            cbase = jnp.int32(cb * N_NODES)
            pbase = jnp.int32(p * NH)

            @pl.loop(0, CE // 16)
            def _mkidx(i):
                sl = pl.ds(i * 16, 16)
                sv = srcb[sl]
                idxb[sl] = sv + cbase
                dv = dstb[sl] - pbase
                ok = (dv >= 0) & (dv < NH)
                offb[sl] = jnp.where(ok, dv, jnp.int32(NH))

            copies = [
                pltpu.async_copy(
                    xcb_hbm.at[idxb.at[pl.ds(g * 128, 128)]],
                    rows.at[pl.ds(g * 128, 128), :], sem)
                for g in range(CE // 128)
            ]
            for cp in copies:
                cp.wait()

            @pl.loop(0, CE // 16)
            def _rmw(i):
                offv = offb[pl.ds(i * 16, 16)]
                for jj in range(16):
                    r = offv[jj]
                    acc[r, :] = jnp.maximum(acc[r, :], rows[i * 16 + jj, :])

        nrows = NH if p == 0 else (N_NODES - NH)
        pltpu.sync_copy(
            acc.at[pl.ds(0, nrows), :],
            out_hbm.at[q].at[pl.ds(cb * N_NODES + p * NH, nrows), :])


def _segmax_sc(xcb, src, dst):
    mesh = plsc.VectorSubcoreMesh(core_axis_name="c", subcore_axis_name="s")
    kern = pl.kernel(
        _segmax_body,
        out_type=jax.ShapeDtypeStruct((NQ, NCB * N_NODES, 16), jnp.float32),
        mesh=mesh,
        compiler_params=pltpu.CompilerParams(use_tc_tiling_on_sc=False),
        scratch_types=[
            pltpu.VMEM((NH + 1, 16), jnp.float32),
            pltpu.VMEM((CE,), jnp.int32),
            pltpu.VMEM((CE,), jnp.int32),
            pltpu.VMEM((CE,), jnp.int32),
            pltpu.VMEM((CE,), jnp.int32),
            pltpu.VMEM((CE, 16), jnp.float32),
            pltpu.SemaphoreType.DMA,
        ],
    )
    return kern(xcb, src, dst)


def _to_cb(a):
    return a.reshape(N_NODES, NCB, 16).transpose(1, 0, 2).reshape(
        NCB * N_NODES, 16)


def _parts_std(p):
    return p.reshape(NQ, NCB, N_NODES, 16).transpose(0, 2, 1, 3).reshape(
        NQ, N_NODES, D)


def _merge(p_ref):
    agg = jnp.max(p_ref[...], axis=0)
    return jnp.where(agg == NEG_INF, jnp.float32(0.0), agg)


def _dense1_body(x_ref, p_ref, w_ref, b_ref, lnw_ref, lnb_ref, eps_ref,
                 a_ref, o_ref):
    agg = _merge(p_ref)
    h = (1.0 + eps_ref[0, 0]) * x_ref[...] + agg
    h = jnp.dot(h, w_ref[...], preferred_element_type=jnp.float32) + b_ref[...]
    mu = jnp.mean(h, axis=-1, keepdims=True)
    var = jnp.mean((h - mu) ** 2, axis=-1, keepdims=True)
    h = (h - mu) * jax.lax.rsqrt(var + 1e-5) * lnw_ref[...] + lnb_ref[...]
    o_ref[...] = jnp.where(h > 0, h, a_ref[0, 0] * h)


def _dense2_body(h_ref, p_ref, w_ref, b_ref, eps_ref, o_ref):
    t = (1.0 + eps_ref[0, 0]) * h_ref[...] + _merge(p_ref)
    o_ref[...] = jnp.dot(t, w_ref[...], preferred_element_type=jnp.float32) \
        + b_ref[...]


def _dense1(x, parts, W1T, b1, ln_w, ln_b, eps1, prelu_a):
    grid = (N_NODES // ROW_BLK,)
    blk = pl.BlockSpec((ROW_BLK, D), lambda i: (i, 0))
    pblk = pl.BlockSpec((NQ, ROW_BLK, D), lambda i: (0, i, 0))
    full = pl.BlockSpec((D, D), lambda i: (0, 0))
    vec = pl.BlockSpec((1, D), lambda i: (0, 0))
    sca = pl.BlockSpec((1, 1), lambda i: (0, 0))
    return pl.pallas_call(
        _dense1_body,
        grid=grid,
        in_specs=[blk, pblk, full, vec, vec, vec, sca, sca],
        out_specs=blk,
        out_shape=jax.ShapeDtypeStruct((N_NODES, D), jnp.float32),
    )(x, parts, W1T, b1.reshape(1, D), ln_w.reshape(1, D), ln_b.reshape(1, D),
      eps1.reshape(1, 1), prelu_a.reshape(1, 1))


def _dense2(h, parts, W2T, b2, eps2):
    grid = (N_NODES // ROW_BLK,)
    blk = pl.BlockSpec((ROW_BLK, D), lambda i: (i, 0))
    pblk = pl.BlockSpec((NQ, ROW_BLK, D), lambda i: (0, i, 0))
    full = pl.BlockSpec((D, D), lambda i: (0, 0))
    vec = pl.BlockSpec((1, D), lambda i: (0, 0))
    sca = pl.BlockSpec((1, 1), lambda i: (0, 0))
    return pl.pallas_call(
        _dense2_body,
        grid=grid,
        in_specs=[blk, pblk, full, vec, sca],
        out_specs=blk,
        out_shape=jax.ShapeDtypeStruct((N_NODES, D), jnp.float32),
    )(h, parts, W2T, b2.reshape(1, D), eps2.reshape(1, 1))


@jax.jit
def kernel(x, edge_index, W1, b1, eps1, ln_w, ln_b, prelu_a, W2, b2, eps2):
    src = edge_index[0]
    dst = edge_index[1]
    p1 = _parts_std(_segmax_sc(_to_cb(x), src, dst))
    h = _dense1(x, p1, W1.T, b1, ln_w, ln_b, eps1, prelu_a)
    p2 = _parts_std(_segmax_sc(_to_cb(h), src, dst))
    return _dense2(h, p2, W2.T, b2, eps2)


# speedup vs baseline: 1.0035x; 1.0035x over previous
import functools
import jax
import jax.numpy as jnp
from jax import lax
from jax.experimental import pallas as pl
from jax.experimental.pallas import tpu as pltpu
from jax.experimental.pallas import tpu_sc as plsc

N_NODES = 10000
D = 128
E_EDGES = 320000
ROW_BLK = 1000

NCB = 8
NQ = 4
EPQ = E_EDGES // NQ
CE = 640
NCHUNK = EPQ // CE
NH = 5008
NEG_INF = float("-inf")


def _segmax_body(xcb_hbm, src_hbm, dst_hbm, out_hbm,
                 acc, srcb, dstb, idxb, offb, rows, sem):
    cid = lax.axis_index("c")
    sid = lax.axis_index("s")
    wid = sid * 2 + cid
    cb = wid % NCB
    q = wid // NCB

    for p in range(2):
        @pl.loop(0, NH + 1)
        def _init(i):
            acc[i, :] = jnp.full((16,), NEG_INF, jnp.float32)

        @pl.loop(0, NCHUNK)
        def _chunk(c):
            base_e = q * EPQ + c * CE
            pltpu.sync_copy(src_hbm.at[pl.ds(base_e, CE)], srcb)
            pltpu.sync_copy(dst_hbm.at[pl.ds(base_e, CE)], dstb)

            cbase = jnp.int32(cb * N_NODES)
            pbase = jnp.int32(p * NH)

            @pl.loop(0, CE // 16)
            def _mkidx(i):
                sl = pl.ds(i * 16, 16)
                idxb[sl] = srcb[sl] + cbase
                dv = dstb[sl] - pbase
                ok = (dv >= 0) & (dv < NH)
                offb[sl] = jnp.where(ok, dv, jnp.int32(NH))

            copies = [
                pltpu.async_copy(
                    xcb_hbm.at[idxb.at[pl.ds(g * 128, 128)]],
                    rows.at[pl.ds(g * 128, 128), :], sem)
                for g in range(CE // 128)
            ]
            for cp in copies:
                cp.wait()

            @pl.loop(0, CE // 16)
            def _rmw(i):
                offv = offb[pl.ds(i * 16, 16)]
                for jj in range(16):
                    r = offv[jj]

                    @pl.when(r < NH)
                    def _():
                        acc[r, :] = jnp.maximum(acc[r, :], rows[i * 16 + jj, :])

        nrows = NH if p == 0 else (N_NODES - NH)
        pltpu.sync_copy(
            acc.at[pl.ds(0, nrows), :],
            out_hbm.at[q].at[pl.ds(cb * N_NODES + p * NH, nrows), :])


def _segmax_sc(xcb, src, dst):
    mesh = plsc.VectorSubcoreMesh(core_axis_name="c", subcore_axis_name="s")
    kern = pl.kernel(
        _segmax_body,
        out_type=jax.ShapeDtypeStruct((NQ, NCB * N_NODES, 16), jnp.float32),
        mesh=mesh,
        compiler_params=pltpu.CompilerParams(use_tc_tiling_on_sc=False),
        scratch_types=[
            pltpu.VMEM((NH + 1, 16), jnp.float32),
            pltpu.VMEM((CE,), jnp.int32),
            pltpu.VMEM((CE,), jnp.int32),
            pltpu.VMEM((CE,), jnp.int32),
            pltpu.VMEM((CE,), jnp.int32),
            pltpu.VMEM((CE, 16), jnp.float32),
            pltpu.SemaphoreType.DMA,
        ],
    )
    return kern(xcb, src, dst)


def _to_cb(a):
    return a.reshape(N_NODES, NCB, 16).transpose(1, 0, 2).reshape(
        NCB * N_NODES, 16)


def _parts_std(p):
    return p.reshape(NQ, NCB, N_NODES, 16).transpose(0, 2, 1, 3).reshape(
        NQ, N_NODES, D)


def _merge(p_ref):
    agg = jnp.max(p_ref[...], axis=0)
    return jnp.where(agg == NEG_INF, jnp.float32(0.0), agg)


def _dense1_body(x_ref, p_ref, w_ref, b_ref, lnw_ref, lnb_ref, eps_ref,
                 a_ref, o_ref):
    agg = _merge(p_ref)
    h = (1.0 + eps_ref[0, 0]) * x_ref[...] + agg
    h = jnp.dot(h, w_ref[...], preferred_element_type=jnp.float32) + b_ref[...]
    mu = jnp.mean(h, axis=-1, keepdims=True)
    var = jnp.mean((h - mu) ** 2, axis=-1, keepdims=True)
    h = (h - mu) * jax.lax.rsqrt(var + 1e-5) * lnw_ref[...] + lnb_ref[...]
    o_ref[...] = jnp.where(h > 0, h, a_ref[0, 0] * h)


def _dense2_body(h_ref, p_ref, w_ref, b_ref, eps_ref, o_ref):
    t = (1.0 + eps_ref[0, 0]) * h_ref[...] + _merge(p_ref)
    o_ref[...] = jnp.dot(t, w_ref[...], preferred_element_type=jnp.float32) \
        + b_ref[...]


def _dense1(x, parts, W1T, b1, ln_w, ln_b, eps1, prelu_a):
    grid = (N_NODES // ROW_BLK,)
    blk = pl.BlockSpec((ROW_BLK, D), lambda i: (i, 0))
    pblk = pl.BlockSpec((NQ, ROW_BLK, D), lambda i: (0, i, 0))
    full = pl.BlockSpec((D, D), lambda i: (0, 0))
    vec = pl.BlockSpec((1, D), lambda i: (0, 0))
    sca = pl.BlockSpec((1, 1), lambda i: (0, 0))
    return pl.pallas_call(
        _dense1_body,
        grid=grid,
        in_specs=[blk, pblk, full, vec, vec, vec, sca, sca],
        out_specs=blk,
        out_shape=jax.ShapeDtypeStruct((N_NODES, D), jnp.float32),
    )(x, parts, W1T, b1.reshape(1, D), ln_w.reshape(1, D), ln_b.reshape(1, D),
      eps1.reshape(1, 1), prelu_a.reshape(1, 1))


def _dense2(h, parts, W2T, b2, eps2):
    grid = (N_NODES // ROW_BLK,)
    blk = pl.BlockSpec((ROW_BLK, D), lambda i: (i, 0))
    pblk = pl.BlockSpec((NQ, ROW_BLK, D), lambda i: (0, i, 0))
    full = pl.BlockSpec((D, D), lambda i: (0, 0))
    vec = pl.BlockSpec((1, D), lambda i: (0, 0))
    sca = pl.BlockSpec((1, 1), lambda i: (0, 0))
    return pl.pallas_call(
        _dense2_body,
        grid=grid,
        in_specs=[blk, pblk, full, vec, sca],
        out_specs=blk,
        out_shape=jax.ShapeDtypeStruct((N_NODES, D), jnp.float32),
    )(h, parts, W2T, b2.reshape(1, D), eps2.reshape(1, 1))


@jax.jit
def kernel(x, edge_index, W1, b1, eps1, ln_w, ln_b, prelu_a, W2, b2, eps2):
    src = edge_index[0]
    dst = edge_index[1]
    p1 = _parts_std(_segmax_sc(_to_cb(x), src, dst))
    h = _dense1(x, p1, W1.T, b1, ln_w, ln_b, eps1, prelu_a)
    p2 = _parts_std(_segmax_sc(_to_cb(h), src, dst))
    return _dense2(h, p2, W2.T, b2, eps2)
